# baseline (speedup 1.0000x reference)
import jax
import jax.numpy as jnp
from jax.experimental import pallas as pl
from jax.experimental.pallas import tpu as pltpu


def _rpn_body(x_ref, wk_ref, wcls_ref, wloc_ref, bconv_ref, bcls_ref,
              bloc_ref, cls_ref, loc_ref, *, n, wp, margin):
    x = x_ref[0]
    acc = None
    for k in range(9):
        dh, dw = divmod(k, 3)
        s = margin + (dh - 1) * wp + (dw - 1)
        part = jax.lax.dot_general(
            x[s:s + n, :], wk_ref[k],
            dimension_numbers=(((1,), (0,)), ((), ())),
            preferred_element_type=jnp.float32)
        acc = part if acc is None else acc + part
    y1 = jnp.maximum(acc + bconv_ref[...], 0.0)

    cls = jax.lax.dot_general(
        y1, wcls_ref[...], dimension_numbers=(((1,), (0,)), ((), ())),
        preferred_element_type=jnp.float32) + bcls_ref[...]
    loc = jax.lax.dot_general(
        y1, wloc_ref[...], dimension_numbers=(((1,), (0,)), ((), ())),
        preferred_element_type=jnp.float32) + bloc_ref[...]

    a = cls[:, 0:9]
    b = cls[:, 9:18]
    m = jnp.maximum(a, b)
    ea = jnp.exp(a - m)
    eb = jnp.exp(b - m)
    denom = ea + eb
    cls_ref[0] = jnp.concatenate([ea / denom, eb / denom], axis=1)
    loc_ref[0] = loc


def kernel(feats, gt_boxes, im_info, W_conv, b_conv, W_cls, b_cls, W_loc, b_loc):
    B, C, H, W = feats.shape
    Hp, Wp = H + 2, W + 2
    N = Hp * Wp
    M = Wp + 1
    n_cls = W_cls.shape[0]
    n_loc = W_loc.shape[0]

    x = jnp.pad(feats, ((0, 0), (0, 0), (1, 1), (1, 1)))
    x = x.transpose(0, 2, 3, 1).reshape(B, N, C)
    x = jnp.pad(x, ((0, 0), (M, M), (0, 0)))

    wk = W_conv.transpose(2, 3, 1, 0).reshape(9, C, C)
    wcls = W_cls.reshape(n_cls, C).T
    wloc = W_loc.reshape(n_loc, C).T

    import functools
    body = functools.partial(_rpn_body, n=N, wp=Wp, margin=M)
    cls_flat, loc_flat = pl.pallas_call(
        body,
        grid=(B,),
        in_specs=[
            pl.BlockSpec((1, N + 2 * M, C), lambda b: (b, 0, 0)),
            pl.BlockSpec((9, C, C), lambda b: (0, 0, 0)),
            pl.BlockSpec((C, n_cls), lambda b: (0, 0)),
            pl.BlockSpec((C, n_loc), lambda b: (0, 0)),
            pl.BlockSpec((1, C), lambda b: (0, 0)),
            pl.BlockSpec((1, n_cls), lambda b: (0, 0)),
            pl.BlockSpec((1, n_loc), lambda b: (0, 0)),
        ],
        out_specs=[
            pl.BlockSpec((1, N, n_cls), lambda b: (b, 0, 0)),
            pl.BlockSpec((1, N, n_loc), lambda b: (b, 0, 0)),
        ],
        out_shape=[
            jax.ShapeDtypeStruct((B, N, n_cls), jnp.float32),
            jax.ShapeDtypeStruct((B, N, n_loc), jnp.float32),
        ],
        compiler_params=pltpu.CompilerParams(
            dimension_semantics=("arbitrary",)),
    )(x, wk, wcls, wloc, b_conv.reshape(1, C), b_cls.reshape(1, n_cls),
      b_loc.reshape(1, n_loc))

    cls = cls_flat.reshape(B, Hp, Wp, n_cls)[:, 1:H + 1, 1:W + 1, :]
    loc = loc_flat.reshape(B, Hp, Wp, n_loc)[:, 1:H + 1, 1:W + 1, :]
    return (cls.transpose(0, 3, 1, 2), loc.transpose(0, 3, 1, 2))


# speedup vs baseline: 1.1274x; 1.1274x over previous
import jax
import jax.numpy as jnp
from jax.experimental import pallas as pl
from jax.experimental.pallas import tpu as pltpu


def _rpn_body(x_ref, wk_ref, wcls_ref, wloc_ref, bconv_ref, bcls_ref,
              bloc_ref, cls_ref, loc_ref, *, n, wp, margin):
    x = x_ref[0]
    acc = None
    for k in range(9):
        dh, dw = divmod(k, 3)
        s = margin + (dh - 1) * wp + (dw - 1)
        part = jax.lax.dot_general(
            x[s:s + n, :], wk_ref[k],
            dimension_numbers=(((1,), (0,)), ((), ())),
            preferred_element_type=jnp.float32)
        acc = part if acc is None else acc + part
    y1 = jnp.maximum(acc + bconv_ref[...], 0.0)

    cls = jax.lax.dot_general(
        y1, wcls_ref[...], dimension_numbers=(((1,), (0,)), ((), ())),
        preferred_element_type=jnp.float32) + bcls_ref[...]
    loc = jax.lax.dot_general(
        y1, wloc_ref[...], dimension_numbers=(((1,), (0,)), ((), ())),
        preferred_element_type=jnp.float32) + bloc_ref[...]

    a = cls[:, 0:9]
    b = cls[:, 9:18]
    m = jnp.maximum(a, b)
    ea = jnp.exp(a - m)
    eb = jnp.exp(b - m)
    denom = ea + eb
    cls_ref[0] = jnp.concatenate([ea / denom, eb / denom], axis=1)
    loc_ref[0] = loc


def kernel(feats, gt_boxes, im_info, W_conv, b_conv, W_cls, b_cls, W_loc, b_loc):
    B, C, H, W = feats.shape
    Hp, Wp = H + 2, W + 2
    N = Hp * Wp
    M = Wp + 1
    n_cls = W_cls.shape[0]
    n_loc = W_loc.shape[0]

    x = jnp.pad(feats, ((0, 0), (0, 0), (1, 1), (1, 1)))
    x = x.transpose(0, 2, 3, 1).reshape(B, N, C)
    x = jnp.pad(x, ((0, 0), (M, M), (0, 0))).astype(jnp.bfloat16)

    wk = W_conv.transpose(2, 3, 1, 0).reshape(9, C, C).astype(jnp.bfloat16)
    wcls = W_cls.reshape(n_cls, C).T
    wloc = W_loc.reshape(n_loc, C).T

    import functools
    body = functools.partial(_rpn_body, n=N, wp=Wp, margin=M)
    cls_flat, loc_flat = pl.pallas_call(
        body,
        grid=(B,),
        in_specs=[
            pl.BlockSpec((1, N + 2 * M, C), lambda b: (b, 0, 0)),
            pl.BlockSpec((9, C, C), lambda b: (0, 0, 0)),
            pl.BlockSpec((C, n_cls), lambda b: (0, 0)),
            pl.BlockSpec((C, n_loc), lambda b: (0, 0)),
            pl.BlockSpec((1, C), lambda b: (0, 0)),
            pl.BlockSpec((1, n_cls), lambda b: (0, 0)),
            pl.BlockSpec((1, n_loc), lambda b: (0, 0)),
        ],
        out_specs=[
            pl.BlockSpec((1, N, n_cls), lambda b: (b, 0, 0)),
            pl.BlockSpec((1, N, n_loc), lambda b: (b, 0, 0)),
        ],
        out_shape=[
            jax.ShapeDtypeStruct((B, N, n_cls), jnp.float32),
            jax.ShapeDtypeStruct((B, N, n_loc), jnp.float32),
        ],
        compiler_params=pltpu.CompilerParams(
            dimension_semantics=("arbitrary",)),
    )(x, wk, wcls, wloc, b_conv.reshape(1, C), b_cls.reshape(1, n_cls),
      b_loc.reshape(1, n_loc))

    cls = cls_flat.reshape(B, Hp, Wp, n_cls)[:, 1:H + 1, 1:W + 1, :]
    loc = loc_flat.reshape(B, Hp, Wp, n_loc)[:, 1:H + 1, 1:W + 1, :]
    return (cls.transpose(0, 3, 1, 2), loc.transpose(0, 3, 1, 2))


# speedup vs baseline: 1.1865x; 1.0524x over previous
import functools

import jax
import jax.numpy as jnp
from jax.experimental import pallas as pl
from jax.experimental.pallas import tpu as pltpu


def _rpn_body(x_ref, wk_ref, wcls_ref, wloc_ref, bconv_ref, bcls_ref,
              bloc_ref, cls_ref, loc_ref, *, n, wp, margin):
    x = x_ref[0]
    acc = None
    for k in range(9):
        dh, dw = divmod(k, 3)
        s = margin + (dh - 1) * wp + (dw - 1)
        part = jax.lax.dot_general(
            wk_ref[k], x[:, s:s + n],
            dimension_numbers=(((1,), (0,)), ((), ())),
            preferred_element_type=jnp.float32)
        acc = part if acc is None else acc + part
    y1 = jnp.maximum(acc + bconv_ref[...], 0.0)
    y1 = y1.astype(jnp.bfloat16)

    cls = jax.lax.dot_general(
        wcls_ref[...], y1, dimension_numbers=(((1,), (0,)), ((), ())),
        preferred_element_type=jnp.float32) + bcls_ref[...]
    loc = jax.lax.dot_general(
        wloc_ref[...], y1, dimension_numbers=(((1,), (0,)), ((), ())),
        preferred_element_type=jnp.float32) + bloc_ref[...]

    a = cls[0:9, :]
    b = cls[9:18, :]
    m = jnp.maximum(a, b)
    ea = jnp.exp(a - m)
    eb = jnp.exp(b - m)
    denom = ea + eb
    cls_ref[0] = jnp.concatenate([ea / denom, eb / denom], axis=0)
    loc_ref[0] = loc


def kernel(feats, gt_boxes, im_info, W_conv, b_conv, W_cls, b_cls, W_loc, b_loc):
    B, C, H, W = feats.shape
    Hp, Wp = H + 2, W + 2
    N = Hp * Wp
    M = Wp + 1
    n_cls = W_cls.shape[0]
    n_loc = W_loc.shape[0]

    x = feats.astype(jnp.bfloat16)
    x = jnp.pad(x, ((0, 0), (0, 0), (1, 1), (1, 1)))
    x = x.reshape(B, C, N)
    x = jnp.pad(x, ((0, 0), (0, 0), (M, M)))

    wk = W_conv.transpose(2, 3, 0, 1).reshape(9, C, C).astype(jnp.bfloat16)
    wcls = W_cls.reshape(n_cls, C).astype(jnp.bfloat16)
    wloc = W_loc.reshape(n_loc, C).astype(jnp.bfloat16)

    body = functools.partial(_rpn_body, n=N, wp=Wp, margin=M)
    cls_flat, loc_flat = pl.pallas_call(
        body,
        grid=(B,),
        in_specs=[
            pl.BlockSpec((1, C, N + 2 * M), lambda b: (b, 0, 0)),
            pl.BlockSpec((9, C, C), lambda b: (0, 0, 0)),
            pl.BlockSpec((n_cls, C), lambda b: (0, 0)),
            pl.BlockSpec((n_loc, C), lambda b: (0, 0)),
            pl.BlockSpec((C, 1), lambda b: (0, 0)),
            pl.BlockSpec((n_cls, 1), lambda b: (0, 0)),
            pl.BlockSpec((n_loc, 1), lambda b: (0, 0)),
        ],
        out_specs=[
            pl.BlockSpec((1, n_cls, N), lambda b: (b, 0, 0)),
            pl.BlockSpec((1, n_loc, N), lambda b: (b, 0, 0)),
        ],
        out_shape=[
            jax.ShapeDtypeStruct((B, n_cls, N), jnp.float32),
            jax.ShapeDtypeStruct((B, n_loc, N), jnp.float32),
        ],
        compiler_params=pltpu.CompilerParams(
            dimension_semantics=("arbitrary",)),
    )(x, wk, wcls, wloc, b_conv.reshape(C, 1), b_cls.reshape(n_cls, 1),
      b_loc.reshape(n_loc, 1))

    cls = cls_flat.reshape(B, n_cls, Hp, Wp)[:, :, 1:H + 1, 1:W + 1]
    loc = loc_flat.reshape(B, n_loc, Hp, Wp)[:, :, 1:H + 1, 1:W + 1]
    return (cls, loc)


# speedup vs baseline: 1.4456x; 1.2184x over previous
import functools

import jax
import jax.numpy as jnp
from jax.experimental import pallas as pl
from jax.experimental.pallas import tpu as pltpu


def _rpn_body(x_ref, wk_ref, wcls_ref, wloc_ref, bconv_ref, bcls_ref,
              bloc_ref, cls_ref, loc_ref, xl_ref, xm_ref, xr_ref,
              *, n, w, margin):
    xb = x_ref[0].astype(jnp.bfloat16)
    c = xb.shape[0]
    next_ = n + 2 * margin

    zl = jnp.zeros((c, margin), jnp.bfloat16)
    xm_ref[:, 0:margin] = zl
    xm_ref[:, margin + n:next_] = zl
    xm_ref[:, margin:margin + n] = xb

    col = jax.lax.broadcasted_iota(jnp.int32, (1, next_), 1)
    wcol = (col - margin) % w
    xm = xm_ref[...]
    xl_ref[...] = jnp.where(wcol == w - 1, jnp.bfloat16(0), xm)
    xr_ref[...] = jnp.where(wcol == 0, jnp.bfloat16(0), xm)

    acc = None
    for k in range(9):
        dh, dw = divmod(k, 3)
        src = (xl_ref, xm_ref, xr_ref)[dw]
        s = margin + (dh - 1) * w + (dw - 1)
        part = jax.lax.dot_general(
            wk_ref[k], src[:, s:s + n],
            dimension_numbers=(((1,), (0,)), ((), ())),
            preferred_element_type=jnp.float32)
        acc = part if acc is None else acc + part
    y1 = jnp.maximum(acc + bconv_ref[...], 0.0)
    y1 = y1.astype(jnp.bfloat16)

    cls = jax.lax.dot_general(
        wcls_ref[...], y1, dimension_numbers=(((1,), (0,)), ((), ())),
        preferred_element_type=jnp.float32) + bcls_ref[...]
    loc = jax.lax.dot_general(
        wloc_ref[...], y1, dimension_numbers=(((1,), (0,)), ((), ())),
        preferred_element_type=jnp.float32) + bloc_ref[...]

    a = cls[0:9, :]
    b = cls[9:18, :]
    m = jnp.maximum(a, b)
    ea = jnp.exp(a - m)
    eb = jnp.exp(b - m)
    denom = ea + eb
    cls_ref[0] = jnp.concatenate([ea / denom, eb / denom], axis=0)
    loc_ref[0] = loc


def kernel(feats, gt_boxes, im_info, W_conv, b_conv, W_cls, b_cls, W_loc, b_loc):
    B, C, H, W = feats.shape
    N = H * W
    M = W + 1
    n_cls = W_cls.shape[0]
    n_loc = W_loc.shape[0]

    x = feats.reshape(B, C, N)

    wk = W_conv.transpose(2, 3, 0, 1).reshape(9, C, C).astype(jnp.bfloat16)
    wcls = W_cls.reshape(n_cls, C).astype(jnp.bfloat16)
    wloc = W_loc.reshape(n_loc, C).astype(jnp.bfloat16)

    body = functools.partial(_rpn_body, n=N, w=W, margin=M)
    cls_flat, loc_flat = pl.pallas_call(
        body,
        grid=(B,),
        in_specs=[
            pl.BlockSpec((1, C, N), lambda b: (b, 0, 0)),
            pl.BlockSpec((9, C, C), lambda b: (0, 0, 0)),
            pl.BlockSpec((n_cls, C), lambda b: (0, 0)),
            pl.BlockSpec((n_loc, C), lambda b: (0, 0)),
            pl.BlockSpec((C, 1), lambda b: (0, 0)),
            pl.BlockSpec((n_cls, 1), lambda b: (0, 0)),
            pl.BlockSpec((n_loc, 1), lambda b: (0, 0)),
        ],
        out_specs=[
            pl.BlockSpec((1, n_cls, N), lambda b: (b, 0, 0)),
            pl.BlockSpec((1, n_loc, N), lambda b: (b, 0, 0)),
        ],
        out_shape=[
            jax.ShapeDtypeStruct((B, n_cls, N), jnp.float32),
            jax.ShapeDtypeStruct((B, n_loc, N), jnp.float32),
        ],
        scratch_shapes=[
            pltpu.VMEM((C, N + 2 * M), jnp.bfloat16),
            pltpu.VMEM((C, N + 2 * M), jnp.bfloat16),
            pltpu.VMEM((C, N + 2 * M), jnp.bfloat16),
        ],
        compiler_params=pltpu.CompilerParams(
            dimension_semantics=("arbitrary",)),
    )(x, wk, wcls, wloc, b_conv.reshape(C, 1), b_cls.reshape(n_cls, 1),
      b_loc.reshape(n_loc, 1))

    return (cls_flat.reshape(B, n_cls, H, W), loc_flat.reshape(B, n_loc, H, W))
